# baseline (speedup 1.0000x reference)
import functools

import jax
import jax.numpy as jnp
from jax import lax
from jax.experimental import pallas as pl
from jax.experimental.pallas import tpu as pltpu
from jax.experimental.pallas import tpu_sc as plsc

K = 8192
D = 256
BETA = 0.25
N = 8192
NC, NS, L = 2, 16, 16
NW = NC * NS
B_PER_W = N // NW
CHUNK = 32
NCHUNK = B_PER_W // CHUNK
HR = 2 * CHUNK
NBUF = 4

_mesh = plsc.VectorSubcoreMesh(
    core_axis_name="c", subcore_axis_name="s", num_cores=NC, num_subcores=NS)


@functools.partial(
    pl.kernel,
    out_type=(
        jax.ShapeDtypeStruct((2 * N, 128), jnp.float32),
        jax.ShapeDtypeStruct((N,), jnp.float32),
        jax.ShapeDtypeStruct((N,), jnp.int32),
    ),
    mesh=_mesh,
    scratch_types=[
        pltpu.VMEM((B_PER_W,), jnp.int32),
        pltpu.VMEM((2 * B_PER_W,), jnp.int32),
        pltpu.VMEM((NBUF, HR, 128), jnp.float32),
        pltpu.VMEM((NBUF, HR, 128), jnp.float32),
        pltpu.VMEM((B_PER_W,), jnp.float32),
        [pltpu.SemaphoreType.DMA] * NBUF,
        [pltpu.SemaphoreType.DMA] * NBUF,
        [pltpu.SemaphoreType.DMA] * NBUF,
        pltpu.SemaphoreType.DMA,
    ],
    compiler_params=pltpu.CompilerParams(
        use_tc_tiling_on_sc=False, needs_layout_passes=False),
)
def _vq_sc(idx_hbm, lat_hbm, w_hbm, q_hbm, loss_hbm, idxo_hbm,
           idx_v, idx2_v, rows4, lat4, loss_v, sgs, sls, sos, sio):
    wid = lax.axis_index("s") * NC + lax.axis_index("c")
    base = wid * B_PER_W
    rbase = 2 * base
    pltpu.sync_copy(idx_hbm.at[pl.ds(base, B_PER_W)], idx_v)
    idxoc = pltpu.async_copy(idx_v, idxo_hbm.at[pl.ds(base, B_PER_W)], sio)
    scale = jnp.float32((1.0 + BETA) / D)
    lane = lax.iota(jnp.int32, L)
    masks = [(lane & d) == 0 for d in (1, 2, 4, 8)]

    def lat_issue(c):
        b = c % NBUF
        return pltpu.async_copy(
            lat_hbm.at[pl.ds(rbase + c * HR, HR)], lat4.at[b], sls[b])

    def gat_issue(c):
        b = c % NBUF
        return pltpu.async_copy(
            w_hbm.at[idx2_v.at[pl.ds(c * HR, HR)]], rows4.at[b], sgs[b])

    lat_pend = [lat_issue(0), lat_issue(1)]

    @plsc.parallel_loop(0, B_PER_W // L)
    def expand(g16):
        k = idx_v[pl.ds(g16 * L, L)]
        i0 = ((k >> 3) << 4) | (k & 7)
        j = g16 * L + lane
        p0 = ((j >> 3) << 4) | (j & 7)
        plsc.store_scatter(idx2_v, [p0], i0)
        plsc.store_scatter(idx2_v, [p0 | 8], i0 | 8)

    gat_pend = [gat_issue(0), gat_issue(1)]
    outs = [None] * NBUF
    for c in range(NCHUNK):
        b = c % NBUF
        gat_pend[c % 2].wait()
        lat_pend[c % 2].wait()
        if c + 2 < NCHUNK:
            nb = (c + 2) % NBUF
            if outs[nb] is not None:
                outs[nb].wait()
                outs[nb] = None
            lat_pend[c % 2] = lat_issue(c + 2)
            gat_pend[c % 2] = gat_issue(c + 2)
        outs[b] = pltpu.async_copy(
            rows4.at[b], q_hbm.at[pl.ds(rbase + c * HR, HR)], sos[b])

        @plsc.parallel_loop(0, CHUNK // L)
        def group_body(g):
            accs = []
            for u in range(L):
                a0 = jnp.zeros((L,), jnp.float32)
                a1 = jnp.zeros((L,), jnp.float32)
                for cc in range(2):
                    r = 32 * g + 16 * (u >> 3) + (u & 7) + 8 * cc
                    for j in range(128 // L):
                        e = (rows4[b, r, pl.ds(j * L, L)]
                             - lat4[b, r, pl.ds(j * L, L)])
                        if j % 2 == 0:
                            a0 = a0 + e * e
                        else:
                            a1 = a1 + e * e
                accs.append(a0 + a1)
            for si, dist in enumerate((1, 2, 4, 8)):
                nxt = []
                for p in range(0, len(accs), 2):
                    x, y = accs[p], accs[p + 1]
                    px = x.at[lane ^ dist].get(mode="promise_in_bounds")
                    py = y.at[lane ^ dist].get(mode="promise_in_bounds")
                    nxt.append(jnp.where(masks[si], x + px, y + py))
                accs = nxt
            loss_v[pl.ds(c * CHUNK + g * L, L)] = accs[0] * scale

    for o in outs:
        if o is not None:
            o.wait()
    idxoc.wait()
    pltpu.sync_copy(loss_v, loss_hbm.at[pl.ds(base, B_PER_W)])


def kernel(gold_encoding_inds, latents, epc, W):
    bs, t, d = latents.shape
    n = bs * t
    idx = gold_encoding_inds[:, 0].astype(jnp.int32)
    w_r = W.reshape(K // 8, 8, 2, 128).transpose(0, 2, 1, 3).reshape(2 * K, 128)
    lat_r = latents.reshape(bs, t // 8, 8, 2, 128).transpose(0, 1, 3, 2, 4)
    lat_r = lat_r.reshape(2 * n, 128)
    q_r, loss, idx_out = _vq_sc(idx, lat_r, w_r)
    gold_quantized = (q_r.reshape(bs, t // 8, 2, 8, 128)
                      .transpose(0, 1, 3, 2, 4).reshape(bs, t, d))
    vq_loss = loss.reshape(bs, t)
    inds_t = idx_out.astype(gold_encoding_inds.dtype).reshape(1, n)
    return gold_quantized, vq_loss, inds_t


# speedup vs baseline: 1.5440x; 1.5440x over previous
import functools

import jax
import jax.numpy as jnp
from jax import lax
from jax.experimental import pallas as pl
from jax.experimental.pallas import tpu as pltpu
from jax.experimental.pallas import tpu_sc as plsc

K = 8192
D = 256
BETA = 0.25
N = 8192
T = 1024
NC, NS, L = 2, 16, 16
NW = NC * NS
B_PER_W = N // NW
CHUNK = 64
NCHUNK = B_PER_W // CHUNK
HR = 2 * CHUNK

_mesh = plsc.VectorSubcoreMesh(
    core_axis_name="c", subcore_axis_name="s", num_cores=NC, num_subcores=NS)


@functools.partial(
    pl.kernel,
    out_type=(
        jax.ShapeDtypeStruct((2 * N, 128), jnp.float32),
        jax.ShapeDtypeStruct((N // 128, 128), jnp.float32),
        jax.ShapeDtypeStruct((N,), jnp.int32),
    ),
    mesh=_mesh,
    scratch_types=[
        pltpu.VMEM((B_PER_W,), jnp.int32),
        pltpu.VMEM((2 * B_PER_W,), jnp.int32),
        pltpu.VMEM((2, HR, 128), jnp.float32),
        pltpu.VMEM((2, HR, 128), jnp.float32),
        pltpu.VMEM((B_PER_W,), jnp.float32),
        pltpu.VMEM((L, L), jnp.float32),
        [pltpu.SemaphoreType.DMA] * 2,
        [pltpu.SemaphoreType.DMA] * 2,
        [pltpu.SemaphoreType.DMA] * 2,
        pltpu.SemaphoreType.DMA,
    ],
    compiler_params=pltpu.CompilerParams(
        use_tc_tiling_on_sc=False, needs_layout_passes=False),
)
def _vq_sc(idx_hbm, lat_hbm, w_hbm, q_hbm, loss_hbm, idxo_hbm,
           idx_v, idx2_v, rows2, lat2, loss_v, trans_v, sgs, sls, sos, sio):
    wid = lax.axis_index("s") * NC + lax.axis_index("c")
    base = wid * B_PER_W
    rbase = 2 * base
    pltpu.sync_copy(idx_hbm.at[pl.ds(base, B_PER_W)], idx_v)
    idxoc = pltpu.async_copy(idx_v, idxo_hbm.at[pl.ds(base, B_PER_W)], sio)
    scale = jnp.float32((1.0 + BETA) / D)
    lane = lax.iota(jnp.int32, L)
    masks = [(lane & d) == 0 for d in (1, 2, 4, 8)]

    def lat_issue(c):
        b = c % 2
        return pltpu.async_copy(
            lat_hbm.at[pl.ds(rbase + c * HR, HR)], lat2.at[b], sls[b])

    def gat_issue(c):
        b = c % 2
        return pltpu.async_copy(
            w_hbm.at[idx2_v.at[pl.ds(c * HR, HR)]], rows2.at[b], sgs[b])

    lat_pend = [lat_issue(0), None]

    @plsc.parallel_loop(0, B_PER_W // L)
    def expand(g16):
        k = idx_v[pl.ds(g16 * L, L)]
        i0 = ((k >> 3) << 4) | (k & 7)
        j = g16 * L + lane
        p0 = ((j >> 3) << 4) | (j & 7)
        plsc.store_scatter(idx2_v, [p0], i0)
        plsc.store_scatter(idx2_v, [p0 | 8], i0 | 8)

    gat_pend = [gat_issue(0), None]
    outs = [None, None]
    for c in range(NCHUNK):
        b = c % 2
        gat_pend[b].wait()
        lat_pend[b].wait()
        if c + 1 < NCHUNK:
            nb = (c + 1) % 2
            if outs[nb] is not None:
                outs[nb].wait()
                outs[nb] = None
            lat_pend[nb] = lat_issue(c + 1)
            gat_pend[nb] = gat_issue(c + 1)
        outs[b] = pltpu.async_copy(
            rows2.at[b], q_hbm.at[pl.ds(rbase + c * HR, HR)], sos[b])

        def group_body(g, carry):
            @plsc.parallel_loop(0, L)
            def tok_body(u):
                a0 = jnp.zeros((L,), jnp.float32)
                a1 = jnp.zeros((L,), jnp.float32)
                r0 = 32 * g + 16 * (u >> 3) + (u & 7)
                for cc in range(2):
                    r = r0 + 8 * cc
                    for j in range(128 // L):
                        e = (rows2[b, r, pl.ds(j * L, L)]
                             - lat2[b, r, pl.ds(j * L, L)])
                        if j % 2 == 0:
                            a0 = a0 + e * e
                        else:
                            a1 = a1 + e * e
                trans_v[u] = a0 + a1

            accs = [trans_v[u] for u in range(L)]
            for si, dist in enumerate((1, 2, 4, 8)):
                nxt = []
                for p in range(0, len(accs), 2):
                    x, y = accs[p], accs[p + 1]
                    px = x.at[lane ^ dist].get(mode="promise_in_bounds")
                    py = y.at[lane ^ dist].get(mode="promise_in_bounds")
                    nxt.append(jnp.where(masks[si], x + px, y + py))
                accs = nxt
            loss_v[pl.ds(c * CHUNK + g * L, L)] = accs[0] * scale
            return carry

        lax.fori_loop(0, CHUNK // L, group_body, 0)
    for o in outs:
        if o is not None:
            o.wait()
    idxoc.wait()
    bb = wid // (T // B_PER_W)
    t0 = (wid % (T // B_PER_W)) * B_PER_W
    pltpu.sync_copy(loss_v.at[pl.ds(0, 128)],
                    loss_hbm.at[(t0 // 128) * (N // T) + bb])
    pltpu.sync_copy(loss_v.at[pl.ds(128, 128)],
                    loss_hbm.at[(t0 // 128 + 1) * (N // T) + bb])


def kernel(gold_encoding_inds, latents, epc, W):
    bs, t, d = latents.shape
    n = bs * t
    idx = gold_encoding_inds[:, 0].astype(jnp.int32)
    w_r = W.reshape(K // 8, 8, 2, 128).transpose(0, 2, 1, 3).reshape(2 * K, 128)
    lat_r = latents.reshape(bs, t // 8, 8, 2, 128).transpose(0, 1, 3, 2, 4)
    lat_r = lat_r.reshape(2 * n, 128)
    q_r, loss_t, idx_out = _vq_sc(idx, lat_r, w_r)
    gold_quantized = (q_r.reshape(bs, t // 8, 2, 8, 128)
                      .transpose(0, 1, 3, 2, 4).reshape(bs, t, d))
    vq_loss = (loss_t.reshape(t // 128, bs, 128)
               .transpose(1, 0, 2).reshape(bs, t))
    inds_t = idx_out.astype(gold_encoding_inds.dtype).reshape(1, n)
    return gold_quantized, vq_loss, inds_t
